# baseline (speedup 1.0000x reference)
import functools

import jax
import jax.numpy as jnp
from jax import lax
from jax.experimental import pallas as pl
from jax.experimental.pallas import tpu as pltpu
from jax.experimental.pallas import tpu_sc as plsc

_B = 16384
_F = 32
_D = 128
_NC, _NS = 2, 16
_NW = _NC * _NS
_BPW = _B // _NW
_CH = 128
_NCHUNK = _BPW // _CH


def _gather_body(user_h, item_h, eug_h, eig_h, eum_h, eim_h,
                 ug_o, ig_o, um_o, im_o,
                 uidx, iidx, ug_v, ig_v, um_v, im_v, sem):
    wid = lax.axis_index("s") * _NC + lax.axis_index("c")
    base = wid * _BPW
    for c in range(_NCHUNK):
        off = base + c * _CH
        pltpu.sync_copy(user_h.at[pl.ds(off, _CH)], uidx)
        pltpu.sync_copy(item_h.at[pl.ds(off, _CH)], iidx)
        d1 = pltpu.async_copy(eug_h.at[uidx], ug_v, sem)
        d2 = pltpu.async_copy(eig_h.at[iidx], ig_v, sem)
        d3 = pltpu.async_copy(eum_h.at[uidx], um_v, sem)
        d4 = pltpu.async_copy(eim_h.at[iidx], im_v, sem)
        d1.wait()
        d2.wait()
        d3.wait()
        d4.wait()
        pltpu.sync_copy(ug_v, ug_o.at[pl.ds(off, _CH)])
        pltpu.sync_copy(ig_v, ig_o.at[pl.ds(off, _CH)])
        pltpu.sync_copy(um_v, um_o.at[pl.ds(off, _CH)])
        pltpu.sync_copy(im_v, im_o.at[pl.ds(off, _CH)])


@functools.lru_cache(maxsize=None)
def _build_sc_gather():
    return pl.kernel(
        _gather_body,
        out_type=(
            jax.ShapeDtypeStruct((_B, _F), jnp.float32),
            jax.ShapeDtypeStruct((_B, _F), jnp.float32),
            jax.ShapeDtypeStruct((_B, _D), jnp.float32),
            jax.ShapeDtypeStruct((_B, _D), jnp.float32),
        ),
        mesh=plsc.VectorSubcoreMesh(core_axis_name="c", subcore_axis_name="s",
                                    num_cores=_NC, num_subcores=_NS),
        scratch_types=[
            pltpu.VMEM((_CH,), jnp.int32),
            pltpu.VMEM((_CH,), jnp.int32),
            pltpu.VMEM((_CH, _F), jnp.float32),
            pltpu.VMEM((_CH, _F), jnp.float32),
            pltpu.VMEM((_CH, _D), jnp.float32),
            pltpu.VMEM((_CH, _D), jnp.float32),
            pltpu.SemaphoreType.DMA,
        ],
        compiler_params=pltpu.CompilerParams(use_tc_tiling_on_sc=False),
    )

_BLK = 2048


def _mlp_body(ug, ig, um, im, w1a, w1b, b1, w2, b2, w3, b3, wpg, wpm, bp, out):
    h = jnp.dot(um[...], w1a[...], preferred_element_type=jnp.float32)
    h += jnp.dot(im[...], w1b[...], preferred_element_type=jnp.float32)
    h = jnp.maximum(h + b1[...], 0.0)
    h = jnp.maximum(
        jnp.dot(h, w2[...], preferred_element_type=jnp.float32) + b2[...], 0.0)
    h = jnp.maximum(
        jnp.dot(h, w3[...], preferred_element_type=jnp.float32) + b3[...], 0.0)
    gmf = ug[...] * ig[...]
    out[...] = (jnp.sum(gmf * wpg[...], axis=1)
                + jnp.sum(h * wpm[...], axis=1) + bp[0, 0])


def _full(shape):
    return pl.BlockSpec(shape, lambda i: (0,) * len(shape))


_mlp = pl.pallas_call(
    _mlp_body,
    grid=(_B // _BLK,),
    in_specs=[
        pl.BlockSpec((_BLK, _F), lambda i: (i, 0)),
        pl.BlockSpec((_BLK, _F), lambda i: (i, 0)),
        pl.BlockSpec((_BLK, _D), lambda i: (i, 0)),
        pl.BlockSpec((_BLK, _D), lambda i: (i, 0)),
        _full((_D, _D)),
        _full((_D, _D)),
        _full((1, _D)),
        _full((_D, 64)),
        _full((1, 64)),
        _full((64, _F)),
        _full((1, _F)),
        _full((1, _F)),
        _full((1, _F)),
        _full((1, 1)),
    ],
    out_specs=pl.BlockSpec((_BLK,), lambda i: (i,)),
    out_shape=jax.ShapeDtypeStruct((_B,), jnp.float32),
)


@jax.jit
def kernel(user, item, eug, eig, eum, eim, W1, b1, W2, b2, W3, b3, Wp, bp):
    user = user.astype(jnp.int32)
    item = item.astype(jnp.int32)
    ug, ig, um, im = _build_sc_gather()(user, item, eug, eig, eum, eim)
    w1a = W1[:_D, :]
    w1b = W1[_D:, :]
    wpg = Wp[:_F, 0].reshape(1, _F)
    wpm = Wp[_F:, 0].reshape(1, _F)
    return _mlp(ug, ig, um, im, w1a, w1b, b1.reshape(1, _D),
                W2, b2.reshape(1, 64), W3, b3.reshape(1, _F),
                wpg, wpm, bp.reshape(1, 1))


# speedup vs baseline: 1.0094x; 1.0094x over previous
import functools

import jax
import jax.numpy as jnp
from jax import lax
from jax.experimental import pallas as pl
from jax.experimental.pallas import tpu as pltpu
from jax.experimental.pallas import tpu_sc as plsc

_B = 16384
_F = 32
_D = 128
_NC, _NS = 2, 16
_NW = _NC * _NS
_BPW = _B // _NW
_CH = 128
_NCHUNK = _BPW // _CH
_L = 16


def _gather_body(user_h, item_h, eug_h, eig_h, eum_h, eim_h,
                 gmf_o, um_o, im_o,
                 uidx, iidx, tu, ti, um_v, im_v, ugt, igt, gmf_v, sem):
    wid = lax.axis_index("s") * _NC + lax.axis_index("c")
    base = wid * _BPW
    pltpu.sync_copy(user_h.at[pl.ds(base, _BPW)], uidx)
    pltpu.sync_copy(item_h.at[pl.ds(base, _BPW)], iidx)

    for g in range(_BPW // _L):
        u16 = uidx[pl.ds(g * _L, _L)]
        i16 = iidx[pl.ds(g * _L, _L)]
        tu[pl.ds(g * _L, _L)] = lax.shift_right_logical(u16, 2)
        ti[pl.ds(g * _L, _L)] = lax.shift_right_logical(i16, 2)

    for c in range(_NCHUNK):
        off = c * _CH
        d1 = pltpu.async_copy(eum_h.at[uidx.at[pl.ds(off, _CH)]], um_v, sem)
        d2 = pltpu.async_copy(eim_h.at[iidx.at[pl.ds(off, _CH)]], im_v, sem)
        d3 = pltpu.async_copy(eug_h.at[tu.at[pl.ds(off, _CH)]], ugt, sem)
        d4 = pltpu.async_copy(eig_h.at[ti.at[pl.ds(off, _CH)]], igt, sem)
        d1.wait()
        d2.wait()
        pltpu.sync_copy(um_v, um_o.at[pl.ds(base + off, _CH)])
        pltpu.sync_copy(im_v, im_o.at[pl.ds(base + off, _CH)])
        d3.wait()
        d4.wait()
        for g in range(_CH // _L):
            rows = lax.iota(jnp.int32, _L) + g * _L
            su = (uidx[pl.ds(off + g * _L, _L)] & 3) * _F
            si = (iidx[pl.ds(off + g * _L, _L)] & 3) * _F

            def body(f, carry):
                uv = plsc.load_gather(ugt, [rows, su + f])
                iv = plsc.load_gather(igt, [rows, si + f])
                plsc.store_scatter(gmf_v,
                                   [jnp.full((_L,), f, jnp.int32),
                                    rows + off],
                                   uv * iv)
                return carry

            lax.fori_loop(0, _F, body, 0)

    pltpu.sync_copy(gmf_v, gmf_o.at[:, pl.ds(base, _BPW)])


@functools.lru_cache(maxsize=None)
def _build_sc_gather():
    return pl.kernel(
        _gather_body,
        out_type=(
            jax.ShapeDtypeStruct((_F, _B), jnp.float32),
            jax.ShapeDtypeStruct((_B, _D), jnp.float32),
            jax.ShapeDtypeStruct((_B, _D), jnp.float32),
        ),
        mesh=plsc.VectorSubcoreMesh(core_axis_name="c", subcore_axis_name="s",
                                    num_cores=_NC, num_subcores=_NS),
        scratch_types=[
            pltpu.VMEM((_BPW,), jnp.int32),
            pltpu.VMEM((_BPW,), jnp.int32),
            pltpu.VMEM((_BPW,), jnp.int32),
            pltpu.VMEM((_BPW,), jnp.int32),
            pltpu.VMEM((_CH, _D), jnp.float32),
            pltpu.VMEM((_CH, _D), jnp.float32),
            pltpu.VMEM((_CH, _D), jnp.float32),
            pltpu.VMEM((_CH, _D), jnp.float32),
            pltpu.VMEM((_F, _BPW), jnp.float32),
            pltpu.SemaphoreType.DMA,
        ],
        compiler_params=pltpu.CompilerParams(needs_layout_passes=False),
    )


_BLK = 2048


def _mlp_body(gmf, um, im, w1a, w1b, b1, w2, b2, w3, b3, wpg, wpm, bp, out):
    h = jnp.dot(um[...], w1a[...], preferred_element_type=jnp.float32)
    h += jnp.dot(im[...], w1b[...], preferred_element_type=jnp.float32)
    h = jnp.maximum(h + b1[...], 0.0)
    h = jnp.maximum(
        jnp.dot(h, w2[...], preferred_element_type=jnp.float32) + b2[...], 0.0)
    h = jnp.maximum(
        jnp.dot(h, w3[...], preferred_element_type=jnp.float32) + b3[...], 0.0)
    out[...] = (jnp.sum(gmf[...] * wpg[...], axis=0)
                + jnp.sum(h * wpm[...], axis=1) + bp[0, 0])


def _full(shape):
    return pl.BlockSpec(shape, lambda i: (0,) * len(shape))


_mlp = pl.pallas_call(
    _mlp_body,
    grid=(_B // _BLK,),
    in_specs=[
        pl.BlockSpec((_F, _BLK), lambda i: (0, i)),
        pl.BlockSpec((_BLK, _D), lambda i: (i, 0)),
        pl.BlockSpec((_BLK, _D), lambda i: (i, 0)),
        _full((_D, _D)),
        _full((_D, _D)),
        _full((1, _D)),
        _full((_D, 64)),
        _full((1, 64)),
        _full((64, _F)),
        _full((1, _F)),
        _full((_F, 1)),
        _full((1, _F)),
        _full((1, 1)),
    ],
    out_specs=pl.BlockSpec((_BLK,), lambda i: (i,)),
    out_shape=jax.ShapeDtypeStruct((_B,), jnp.float32),
)


@jax.jit
def kernel(user, item, eug, eig, eum, eim, W1, b1, W2, b2, W3, b3, Wp, bp):
    user = user.astype(jnp.int32)
    item = item.astype(jnp.int32)
    eug4 = eug.reshape(-1, _D)
    eig4 = eig.reshape(-1, _D)
    gmfT, um, im = _build_sc_gather()(user, item, eug4, eig4, eum, eim)
    w1a = W1[:_D, :]
    w1b = W1[_D:, :]
    wpg = Wp[:_F, :]
    wpm = Wp[_F:, 0].reshape(1, _F)
    return _mlp(gmfT, um, im, w1a, w1b, b1.reshape(1, _D),
                W2, b2.reshape(1, 64), W3, b3.reshape(1, _F),
                wpg, wpm, bp.reshape(1, 1))


# speedup vs baseline: 7.4496x; 7.3802x over previous
import functools

import jax
import jax.numpy as jnp
from jax import lax
from jax.experimental import pallas as pl
from jax.experimental.pallas import tpu as pltpu
from jax.experimental.pallas import tpu_sc as plsc

_B = 16384
_F = 32
_D = 128
_NC, _NS = 2, 16
_NW = _NC * _NS
_BPW = _B // _NW
_CH = 128
_NCHUNK = _BPW // _CH


def _gather_body(user_h, item_h, eum_h, eim_h, um_o, im_o,
                 uidx, iidx, um_v, im_v, sem, osem):
    wid = lax.axis_index("s") * _NC + lax.axis_index("c")
    base = wid * _BPW
    pltpu.sync_copy(user_h.at[pl.ds(base, _BPW)], uidx)
    pltpu.sync_copy(item_h.at[pl.ds(base, _BPW)], iidx)

    outs = []
    for c in range(_NCHUNK):
        off = c * _CH
        b = c % 2
        d1 = pltpu.async_copy(eum_h.at[uidx.at[pl.ds(off, _CH)]],
                              um_v.at[b], sem)
        d2 = pltpu.async_copy(eim_h.at[iidx.at[pl.ds(off, _CH)]],
                              im_v.at[b], sem)
        d1.wait()
        d2.wait()
        if len(outs) == 2:
            outs.pop(0).wait()
            outs.pop(0).wait()
        o1 = pltpu.async_copy(um_v.at[b], um_o.at[pl.ds(base + off, _CH)],
                              osem)
        o2 = pltpu.async_copy(im_v.at[b], im_o.at[pl.ds(base + off, _CH)],
                              osem)
        outs += [o1, o2]
    for o in outs:
        o.wait()


@functools.lru_cache(maxsize=None)
def _build_sc_gather():
    return pl.kernel(
        _gather_body,
        out_type=(
            jax.ShapeDtypeStruct((_B, _D), jnp.float32),
            jax.ShapeDtypeStruct((_B, _D), jnp.float32),
        ),
        mesh=plsc.VectorSubcoreMesh(core_axis_name="c", subcore_axis_name="s",
                                    num_cores=_NC, num_subcores=_NS),
        scratch_types=[
            pltpu.VMEM((_BPW,), jnp.int32),
            pltpu.VMEM((_BPW,), jnp.int32),
            pltpu.VMEM((2, _CH, _D), jnp.float32),
            pltpu.VMEM((2, _CH, _D), jnp.float32),
            pltpu.SemaphoreType.DMA,
            pltpu.SemaphoreType.DMA,
        ],
    )


_BLK = 2048


def _mlp_body(ug, ig, um, im, w1a, w1b, b1, w2, b2, w3, b3, wpg, wpm, bp, out):
    h = jnp.dot(um[...], w1a[...], preferred_element_type=jnp.float32)
    h += jnp.dot(im[...], w1b[...], preferred_element_type=jnp.float32)
    h = jnp.maximum(h + b1[...], 0.0)
    h = jnp.maximum(
        jnp.dot(h, w2[...], preferred_element_type=jnp.float32) + b2[...], 0.0)
    h = jnp.maximum(
        jnp.dot(h, w3[...], preferred_element_type=jnp.float32) + b3[...], 0.0)
    gmf = ug[...] * ig[...]
    out[...] = (jnp.sum(gmf * wpg[...], axis=1)
                + jnp.sum(h * wpm[...], axis=1) + bp[0, 0])


def _full(shape):
    return pl.BlockSpec(shape, lambda i: (0,) * len(shape))


_mlp = pl.pallas_call(
    _mlp_body,
    grid=(_B // _BLK,),
    in_specs=[
        pl.BlockSpec((_BLK, _F), lambda i: (i, 0)),
        pl.BlockSpec((_BLK, _F), lambda i: (i, 0)),
        pl.BlockSpec((_BLK, _D), lambda i: (i, 0)),
        pl.BlockSpec((_BLK, _D), lambda i: (i, 0)),
        _full((_D, _D)),
        _full((_D, _D)),
        _full((1, _D)),
        _full((_D, 64)),
        _full((1, 64)),
        _full((64, _F)),
        _full((1, _F)),
        _full((1, _F)),
        _full((1, _F)),
        _full((1, 1)),
    ],
    out_specs=pl.BlockSpec((_BLK,), lambda i: (i,)),
    out_shape=jax.ShapeDtypeStruct((_B,), jnp.float32),
)


@jax.jit
def kernel(user, item, eug, eig, eum, eim, W1, b1, W2, b2, W3, b3, Wp, bp):
    user = user.astype(jnp.int32)
    item = item.astype(jnp.int32)
    um, im = _build_sc_gather()(user, item, eum, eim)
    ug = jnp.take(eug, user, axis=0)
    ig = jnp.take(eig, item, axis=0)
    w1a = W1[:_D, :]
    w1b = W1[_D:, :]
    wpg = Wp[:_F, 0].reshape(1, _F)
    wpm = Wp[_F:, 0].reshape(1, _F)
    return _mlp(ug, ig, um, im, w1a, w1b, b1.reshape(1, _D),
                W2, b2.reshape(1, 64), W3, b3.reshape(1, _F),
                wpg, wpm, bp.reshape(1, 1))


# speedup vs baseline: 8.1973x; 1.1004x over previous
import functools

import jax
import jax.numpy as jnp
from jax import lax
from jax.experimental import pallas as pl
from jax.experimental.pallas import tpu as pltpu
from jax.experimental.pallas import tpu_sc as plsc

_B = 16384
_F = 32
_D = 128
_NC, _NS = 2, 16
_NW = _NC * _NS
_BPW = _B // _NW
_CH = 128
_NCHUNK = _BPW // _CH


def _make_gather_body(bpw):
    nchunk = bpw // _CH

    def _gather_body(user_h, item_h, eum_h, eim_h, um_o, im_o,
                     uidx, iidx, um_v, im_v, sem, osem):
        wid = lax.axis_index("s") * _NC + lax.axis_index("c")
        base = wid * bpw
        pltpu.sync_copy(user_h.at[pl.ds(base, bpw)], uidx)
        pltpu.sync_copy(item_h.at[pl.ds(base, bpw)], iidx)

        def issue_gather(c):
            b = c % 2
            off = c * _CH
            return (
                pltpu.async_copy(eum_h.at[uidx.at[pl.ds(off, _CH)]],
                                 um_v.at[b], sem),
                pltpu.async_copy(eim_h.at[iidx.at[pl.ds(off, _CH)]],
                                 im_v.at[b], sem),
            )

        writes = {}
        gath = issue_gather(0)
        for c in range(nchunk):
            b = c % 2
            off = c * _CH
            for d in gath:
                d.wait()
            w = (
                pltpu.async_copy(um_v.at[b],
                                 um_o.at[pl.ds(base + off, _CH)], osem),
                pltpu.async_copy(im_v.at[b],
                                 im_o.at[pl.ds(base + off, _CH)], osem),
            )
            writes[b] = w
            if c + 1 < nchunk:
                nb_ = (c + 1) % 2
                if nb_ in writes:
                    for d in writes.pop(nb_):
                        d.wait()
                gath = issue_gather(c + 1)
        for w in writes.values():
            for d in w:
                d.wait()

    return _gather_body


@functools.lru_cache(maxsize=None)
def _build_sc_gather(nb):
    bpw = nb // _NW
    return pl.kernel(
        _make_gather_body(bpw),
        out_type=(
            jax.ShapeDtypeStruct((nb, _D), jnp.float32),
            jax.ShapeDtypeStruct((nb, _D), jnp.float32),
        ),
        mesh=plsc.VectorSubcoreMesh(core_axis_name="c", subcore_axis_name="s",
                                    num_cores=_NC, num_subcores=_NS),
        scratch_types=[
            pltpu.VMEM((bpw,), jnp.int32),
            pltpu.VMEM((bpw,), jnp.int32),
            pltpu.VMEM((2, _CH, _D), jnp.float32),
            pltpu.VMEM((2, _CH, _D), jnp.float32),
            pltpu.SemaphoreType.DMA,
            pltpu.SemaphoreType.DMA,
        ],
    )


_BLK = 1024


def _mlp_body(ug, ig, um, im, w1, b1, w2, b2, w3, b3, wp, bp, out):
    w1_ = w1[...]
    h = jnp.dot(um[...], w1_[:_D], preferred_element_type=jnp.float32)
    h += jnp.dot(im[...], w1_[_D:], preferred_element_type=jnp.float32)
    h = jnp.maximum(h + b1[...], 0.0)
    h = jnp.maximum(
        jnp.dot(h, w2[...], preferred_element_type=jnp.float32) + b2[...], 0.0)
    h = jnp.maximum(
        jnp.dot(h, w3[...], preferred_element_type=jnp.float32) + b3[...], 0.0)
    gmf = ug[...] * ig[...]
    wp_ = wp[...]
    out[...] = (jnp.dot(gmf, wp_[:_F], preferred_element_type=jnp.float32)
                + jnp.dot(h, wp_[_F:], preferred_element_type=jnp.float32)
                + bp[0, 0])


def _full(shape):
    return pl.BlockSpec(shape, lambda i: (0,) * len(shape))


@functools.lru_cache(maxsize=None)
def _build_mlp(nb):
    return pl.pallas_call(
        _mlp_body,
        grid=(nb // _BLK,),
        in_specs=[
            pl.BlockSpec((_BLK, _F), lambda i: (i, 0)),
            pl.BlockSpec((_BLK, _F), lambda i: (i, 0)),
            pl.BlockSpec((_BLK, _D), lambda i: (i, 0)),
            pl.BlockSpec((_BLK, _D), lambda i: (i, 0)),
            _full((2 * _D, _D)),
            _full((1, _D)),
            _full((_D, 64)),
            _full((1, 64)),
            _full((64, _F)),
            _full((1, _F)),
            _full((2 * _F, 1)),
            _full((1, 1)),
        ],
        out_specs=pl.BlockSpec((_BLK, 1), lambda i: (i, 0)),
        out_shape=jax.ShapeDtypeStruct((nb, 1), jnp.float32),
    )


@jax.jit
def kernel(user, item, eug, eig, eum, eim, W1, b1, W2, b2, W3, b3, Wp, bp):
    user = user.astype(jnp.int32)
    item = item.astype(jnp.int32)
    ug = eug.at[user].get(mode="promise_in_bounds")
    ig = eig.at[item].get(mode="promise_in_bounds")

    um, im = _build_sc_gather(_B)(user, item, eum, eim)
    out2d = _build_mlp(_B)(ug, ig, um, im, W1, b1.reshape(1, _D),
                           W2, b2.reshape(1, 64), W3, b3.reshape(1, _F),
                           Wp, bp.reshape(1, 1))
    return out2d.reshape(-1)


# speedup vs baseline: 8.5889x; 1.0478x over previous
import functools

import jax
import jax.numpy as jnp
from jax import lax
from jax.experimental import pallas as pl
from jax.experimental.pallas import tpu as pltpu
from jax.experimental.pallas import tpu_sc as plsc

_B = 16384
_F = 32
_D = 128
_NC, _NS = 2, 16
_NW = _NC * _NS
_BPW = _B // _NW
_CH = 128
_NCHUNK = _BPW // _CH


def _make_gather_body(bpw):
    nchunk = bpw // _CH

    def _gather_body(user_h, item_h, eum_h, eim_h, um_o, im_o,
                     uidx, iidx, um_v, im_v, sem, osem):
        wid = lax.axis_index("s") * _NC + lax.axis_index("c")
        base = wid * bpw
        pltpu.sync_copy(user_h.at[pl.ds(base, bpw)], uidx)
        pltpu.sync_copy(item_h.at[pl.ds(base, bpw)], iidx)

        def issue_gather(c):
            b = c % 2
            off = c * _CH
            return (
                pltpu.async_copy(eum_h.at[uidx.at[pl.ds(off, _CH)]],
                                 um_v.at[b], sem),
                pltpu.async_copy(eim_h.at[iidx.at[pl.ds(off, _CH)]],
                                 im_v.at[b], sem),
            )

        writes = {}
        gath = issue_gather(0)
        for c in range(nchunk):
            b = c % 2
            off = c * _CH
            for d in gath:
                d.wait()
            w = (
                pltpu.async_copy(um_v.at[b],
                                 um_o.at[pl.ds(base + off, _CH)], osem),
                pltpu.async_copy(im_v.at[b],
                                 im_o.at[pl.ds(base + off, _CH)], osem),
            )
            writes[b] = w
            if c + 1 < nchunk:
                nb_ = (c + 1) % 2
                if nb_ in writes:
                    for d in writes.pop(nb_):
                        d.wait()
                gath = issue_gather(c + 1)
        for w in writes.values():
            for d in w:
                d.wait()

    return _gather_body


@functools.lru_cache(maxsize=None)
def _build_sc_gather(nb):
    bpw = nb // _NW
    return pl.kernel(
        _make_gather_body(bpw),
        out_type=(
            jax.ShapeDtypeStruct((nb, _D), jnp.float32),
            jax.ShapeDtypeStruct((nb, _D), jnp.float32),
        ),
        mesh=plsc.VectorSubcoreMesh(core_axis_name="c", subcore_axis_name="s",
                                    num_cores=_NC, num_subcores=_NS),
        scratch_types=[
            pltpu.VMEM((bpw,), jnp.int32),
            pltpu.VMEM((bpw,), jnp.int32),
            pltpu.VMEM((2, _CH, _D), jnp.float32),
            pltpu.VMEM((2, _CH, _D), jnp.float32),
            pltpu.SemaphoreType.DMA,
            pltpu.SemaphoreType.DMA,
        ],
    )


_BLK = 2048


def _mlp_body(ug, ig, um, im, w1, b1, w2, b2, w3, b3, wp, bp, out):
    w1_ = w1[...]
    h = jnp.dot(um[...], w1_[:_D], preferred_element_type=jnp.float32)
    h += jnp.dot(im[...], w1_[_D:], preferred_element_type=jnp.float32)
    h = jnp.maximum(h + b1[...], 0.0)
    h = jnp.maximum(
        jnp.dot(h, w2[...], preferred_element_type=jnp.float32) + b2[...], 0.0)
    h = jnp.maximum(
        jnp.dot(h, w3[...], preferred_element_type=jnp.float32) + b3[...], 0.0)
    gmf = ug[...] * ig[...]
    wp_ = wp[...]
    out[...] = (jnp.dot(gmf, wp_[:_F], preferred_element_type=jnp.float32)
                + jnp.dot(h, wp_[_F:], preferred_element_type=jnp.float32)
                + bp[0, 0])


def _full(shape):
    return pl.BlockSpec(shape, lambda i: (0,) * len(shape))


@functools.lru_cache(maxsize=None)
def _build_mlp(nb):
    return pl.pallas_call(
        _mlp_body,
        grid=(nb // _BLK,),
        in_specs=[
            pl.BlockSpec((_BLK, _F), lambda i: (i, 0)),
            pl.BlockSpec((_BLK, _F), lambda i: (i, 0)),
            pl.BlockSpec((_BLK, _D), lambda i: (i, 0)),
            pl.BlockSpec((_BLK, _D), lambda i: (i, 0)),
            _full((2 * _D, _D)),
            _full((1, _D)),
            _full((_D, 64)),
            _full((1, 64)),
            _full((64, _F)),
            _full((1, _F)),
            _full((2 * _F, 1)),
            _full((1, 1)),
        ],
        out_specs=pl.BlockSpec((_BLK, 1), lambda i: (i, 0)),
        out_shape=jax.ShapeDtypeStruct((nb, 1), jnp.float32),
    )


@jax.jit
def kernel(user, item, eug, eig, eum, eim, W1, b1, W2, b2, W3, b3, Wp, bp):
    user = user.astype(jnp.int32)
    item = item.astype(jnp.int32)
    ug = eug.at[user].get(mode="promise_in_bounds")
    ig = eig.at[item].get(mode="promise_in_bounds")

    um, im = _build_sc_gather(_B)(user, item, eum, eim)
    out2d = _build_mlp(_B)(ug, ig, um, im, W1, b1.reshape(1, _D),
                           W2, b2.reshape(1, 64), W3, b3.reshape(1, _F),
                           Wp, bp.reshape(1, 1))
    return out2d.reshape(-1)
